# baseline (speedup 1.0000x reference)
import jax
import jax.numpy as jnp
from jax import lax
from jax.experimental import pallas as pl
from jax.experimental.pallas import tpu as pltpu
from jax.experimental.pallas import tpu_sc as plsc

N_NODES = 10000
N_EDGES = 320000
D = 128
DH = D // 2

N_PAD = 10240
NC, NS = 2, 16
CHUNK = 128
CPT = 160
E_PAD = NS * CPT * CHUNK
ROWS_PER_SUB = N_PAD // NS

_MESH = plsc.VectorSubcoreMesh(core_axis_name="c", subcore_axis_name="s")


NBD = 4
CPTD = CPT // 2


def _deg_body(dst2d, ones_h, zeros_h, degp, dstv, onesv, zerov, degacc, semd):
    c = lax.axis_index("c")
    s = lax.axis_index("s")
    w = c * NS + s
    pltpu.sync_copy(dst2d.at[pl.ds(w * CPTD, CPTD)], dstv)
    pltpu.sync_copy(ones_h, onesv)
    pltpu.sync_copy(zeros_h, zerov)
    pltpu.sync_copy(zerov, degacc.at[pl.ds(s * ROWS_PER_SUB, ROWS_PER_SUB)])
    plsc.subcore_barrier()

    def grp(gi, carry):
        for k in range(NBD):
            j = gi * NBD + k

            @pl.when(j >= NBD)
            def _():
                pltpu.make_async_copy(
                    onesv, degacc.at[dstv.at[j - NBD]], semd.at[k]).wait()

            pltpu.async_copy(onesv, degacc.at[dstv.at[j]], semd.at[k],
                             add=True)
        return carry

    lax.fori_loop(0, CPTD // NBD, grp, 0)
    for k in range(NBD):
        j = CPTD - NBD + k
        pltpu.make_async_copy(onesv, degacc.at[dstv.at[j]], semd.at[k]).wait()
    plsc.subcore_barrier()
    pltpu.sync_copy(degacc.at[pl.ds(s * ROWS_PER_SUB, ROWS_PER_SUB)], zerov)
    pltpu.sync_copy(zerov, degp.at[c, pl.ds(s * ROWS_PER_SUB, ROWS_PER_SUB)])


_deg_call = pl.kernel(
    _deg_body,
    out_type=jax.ShapeDtypeStruct((NC, N_PAD), jnp.float32),
    mesh=_MESH,
    scratch_types=[
        pltpu.VMEM((CPTD, CHUNK), jnp.int32),
        pltpu.VMEM((CHUNK,), jnp.float32),
        pltpu.VMEM((ROWS_PER_SUB,), jnp.float32),
        pltpu.VMEM_SHARED((N_PAD,), jnp.float32),
        pltpu.SemaphoreType.DMA((NBD,)),
    ],
)


NBUF = 4
LAG = 2
IGRP = 16
NIGRP = CPT // IGRP


def _seg_body(g0, g1, src2d, dst2d, z2d_h, p3, srcb, dstb, rowsv, acc, table,
              semg, sems, semis, semid):
    c = lax.axis_index("c")
    s = lax.axis_index("s")
    ebase = s * CPT

    pltpu.sync_copy(z2d_h, rowsv.at[0])
    for r in range(ROWS_PER_SUB // CHUNK):
        sl = pl.ds(s * ROWS_PER_SUB + r * CHUNK, CHUNK)
        pltpu.sync_copy(rowsv.at[0], acc.at[sl])

    @pl.when(c == 0)
    def _():
        for r in range(ROWS_PER_SUB // CHUNK):
            sl = pl.ds(s * ROWS_PER_SUB + r * CHUNK, CHUNK)
            pltpu.sync_copy(g0.at[sl], rowsv.at[1])
            pltpu.sync_copy(rowsv.at[1], table.at[sl])

    @pl.when(c == 1)
    def _():
        for r in range(ROWS_PER_SUB // CHUNK):
            sl = pl.ds(s * ROWS_PER_SUB + r * CHUNK, CHUNK)
            pltpu.sync_copy(g1.at[sl], rowsv.at[1])
            pltpu.sync_copy(rowsv.at[1], table.at[sl])

    plsc.subcore_barrier()

    pltpu.async_copy(src2d.at[pl.ds(ebase, IGRP)], srcb.at[0], semis.at[0])
    pltpu.async_copy(dst2d.at[pl.ds(ebase, IGRP)], dstb.at[0], semid.at[0])
    pltpu.make_async_copy(
        src2d.at[pl.ds(ebase, IGRP)], srcb.at[0], semis.at[0]).wait()
    pltpu.make_async_copy(
        dst2d.at[pl.ds(ebase, IGRP)], dstb.at[0], semid.at[0]).wait()
    for b in range(LAG):
        pltpu.async_copy(table.at[srcb.at[0, b]], rowsv.at[b], semg.at[b])

    def grp(gi, carry):
        bi = lax.rem(gi, 2)
        bn = lax.rem(gi + 1, 2)

        for k in range(IGRP):
            j = gi * IGRP + k
            b = k % NBUF
            bf = (b + LAG) % NBUF
            pltpu.make_async_copy(
                table.at[srcb.at[bi, k]], rowsv.at[b], semg.at[b]).wait()
            pltpu.async_copy(rowsv.at[b], acc.at[dstb.at[bi, k]], sems.at[b],
                             add=True)
            if k >= LAG:
                pltpu.make_async_copy(
                    rowsv.at[bf], acc.at[dstb.at[bi, k - LAG]],
                    sems.at[bf]).wait()
            else:
                @pl.when(j >= LAG)
                def _():
                    pltpu.make_async_copy(
                        rowsv.at[bf], acc.at[dstb.at[bn, k + IGRP - LAG]],
                        sems.at[bf]).wait()

            if k == LAG:
                @pl.when(gi + 1 < NIGRP)
                def _():
                    nb = ebase + (gi + 1) * IGRP
                    pltpu.async_copy(src2d.at[pl.ds(nb, IGRP)], srcb.at[bn],
                                     semis.at[bn])
                    pltpu.async_copy(dst2d.at[pl.ds(nb, IGRP)], dstb.at[bn],
                                     semid.at[bn])

            if k == IGRP - LAG:
                @pl.when(gi + 1 < NIGRP)
                def _():
                    nb = ebase + (gi + 1) * IGRP
                    pltpu.make_async_copy(
                        src2d.at[pl.ds(nb, IGRP)], srcb.at[bn],
                        semis.at[bn]).wait()
                    pltpu.make_async_copy(
                        dst2d.at[pl.ds(nb, IGRP)], dstb.at[bn],
                        semid.at[bn]).wait()

            if k + LAG < IGRP:
                pltpu.async_copy(table.at[srcb.at[bi, k + LAG]],
                                 rowsv.at[bf], semg.at[bf])
            else:
                @pl.when(gi + 1 < NIGRP)
                def _():
                    pltpu.async_copy(table.at[srcb.at[bn, k + LAG - IGRP]],
                                     rowsv.at[bf], semg.at[bf])
        return carry

    lax.fori_loop(0, NIGRP, grp, 0)
    for k in range(IGRP - LAG, IGRP):
        pltpu.make_async_copy(
            rowsv.at[k % NBUF], acc.at[dstb.at[(NIGRP - 1) % 2, k]],
            sems.at[k % NBUF]).wait()

    plsc.subcore_barrier()
    for r in range(ROWS_PER_SUB // CHUNK):
        base = s * ROWS_PER_SUB + r * CHUNK
        pltpu.sync_copy(acc.at[pl.ds(base, CHUNK)], rowsv.at[0])
        pltpu.sync_copy(rowsv.at[0], p3.at[c, pl.ds(base, CHUNK)])


_seg_call = pl.kernel(
    _seg_body,
    out_type=jax.ShapeDtypeStruct((NC, N_PAD, DH), jnp.float32),
    mesh=_MESH,
    scratch_types=[
        pltpu.VMEM((2, IGRP, CHUNK), jnp.int32),
        pltpu.VMEM((2, IGRP, CHUNK), jnp.int32),
        pltpu.VMEM((NBUF, CHUNK, DH), jnp.float32),
        pltpu.VMEM_SHARED((N_PAD, DH), jnp.float32),
        pltpu.VMEM_SHARED((N_PAD, DH), jnp.float32),
        pltpu.SemaphoreType.DMA((NBUF,)),
        pltpu.SemaphoreType.DMA((NBUF,)),
        pltpu.SemaphoreType.DMA((2,)),
        pltpu.SemaphoreType.DMA((2,)),
    ],
    compiler_params=pltpu.CompilerParams(use_tc_tiling_on_sc=False),
)


_RB = 512

def _g_body(xref, wref, degref, g0ref, g1ref):
    deg = jnp.maximum(degref[0] + degref[1], 1.0)
    dinv = lax.rsqrt(deg)
    h = jnp.dot(xref[...], wref[...], preferred_element_type=jnp.float32) * dinv
    g0ref[...] = h[:, :DH]
    g1ref[...] = h[:, DH:]


def _g_call(x_pad, W, degp3):
    return pl.pallas_call(
        _g_body,
        grid=(N_PAD // _RB,),
        in_specs=[
            pl.BlockSpec((_RB, D), lambda i: (i, 0)),
            pl.BlockSpec((D, D), lambda i: (0, 0)),
            pl.BlockSpec((NC, _RB, 1), lambda i: (0, i, 0)),
        ],
        out_specs=[
            pl.BlockSpec((_RB, DH), lambda i: (i, 0)),
            pl.BlockSpec((_RB, DH), lambda i: (i, 0)),
        ],
        out_shape=[
            jax.ShapeDtypeStruct((N_PAD, DH), jnp.float32),
            jax.ShapeDtypeStruct((N_PAD, DH), jnp.float32),
        ],
    )(x_pad, W, degp3)


def _out_body(pref, degref, bref, oref):
    deg = jnp.maximum(degref[0] + degref[1], 1.0)
    dinv = lax.rsqrt(deg)
    s = jnp.concatenate([pref[0], pref[1]], axis=1)
    oref[...] = s * dinv + bref[...]


def _out_call(p3, degp3, b2d):
    return pl.pallas_call(
        _out_body,
        grid=(N_PAD // _RB,),
        in_specs=[
            pl.BlockSpec((NC, _RB, DH), lambda i: (0, i, 0)),
            pl.BlockSpec((NC, _RB, 1), lambda i: (0, i, 0)),
            pl.BlockSpec((1, D), lambda i: (0, 0)),
        ],
        out_specs=pl.BlockSpec((_RB, D), lambda i: (i, 0)),
        out_shape=jax.ShapeDtypeStruct((N_PAD, D), jnp.float32),
    )(p3, degp3, b2d)


@jax.jit
def kernel(t, x, edge_index, W, b):
    del t
    src = edge_index[0].astype(jnp.int32)
    dst = edge_index[1].astype(jnp.int32)
    pad = E_PAD - N_EDGES
    src2d = jnp.pad(src, (0, pad), constant_values=N_NODES).reshape(E_PAD // CHUNK, CHUNK)
    dst2d = jnp.pad(dst, (0, pad), constant_values=N_NODES).reshape(E_PAD // CHUNK, CHUNK)
    x_pad = jnp.pad(x.astype(jnp.float32), ((0, N_PAD - N_NODES), (0, 0)))

    ones_h = jnp.ones((CHUNK,), jnp.float32)
    zeros_h = jnp.zeros((ROWS_PER_SUB,), jnp.float32)
    z2d_h = jnp.zeros((CHUNK, DH), jnp.float32)

    degp = _deg_call(dst2d, ones_h, zeros_h)
    degp3 = degp.reshape(NC, N_PAD, 1)
    g0, g1 = _g_call(x_pad, W.astype(jnp.float32), degp3)
    p3 = _seg_call(g0, g1, src2d, dst2d, z2d_h)
    out = _out_call(p3, degp3, b.reshape(1, D).astype(jnp.float32))
    return out[:N_NODES]


# speedup vs baseline: 1.0131x; 1.0131x over previous
import jax
import jax.numpy as jnp
from jax import lax
from jax.experimental import pallas as pl
from jax.experimental.pallas import tpu as pltpu
from jax.experimental.pallas import tpu_sc as plsc

N_NODES = 10000
N_EDGES = 320000
D = 128
DH = D // 2

N_PAD = 10240
NC, NS = 2, 16
CHUNK = 128
CPT = 160
E_PAD = NS * CPT * CHUNK
ROWS_PER_SUB = N_PAD // NS

_MESH = plsc.VectorSubcoreMesh(core_axis_name="c", subcore_axis_name="s")


NBD = 8
CPTD = CPT // 2


def _deg_body(dst2d, ones_h, zeros_h, degp, dstv, onesv, zerov, degacc, semd):
    c = lax.axis_index("c")
    s = lax.axis_index("s")
    w = c * NS + s
    pltpu.sync_copy(dst2d.at[pl.ds(w * CPTD, CPTD)], dstv)
    pltpu.sync_copy(ones_h, onesv)
    pltpu.sync_copy(zeros_h, zerov)
    pltpu.sync_copy(zerov, degacc.at[pl.ds(s * ROWS_PER_SUB, ROWS_PER_SUB)])
    plsc.subcore_barrier()

    def grp(gi, carry):
        for k in range(NBD):
            j = gi * NBD + k

            @pl.when(j >= NBD)
            def _():
                pltpu.make_async_copy(
                    onesv, degacc.at[dstv.at[j - NBD]], semd.at[k]).wait()

            pltpu.async_copy(onesv, degacc.at[dstv.at[j]], semd.at[k],
                             add=True)
        return carry

    lax.fori_loop(0, CPTD // NBD, grp, 0)
    for k in range(NBD):
        j = CPTD - NBD + k
        pltpu.make_async_copy(onesv, degacc.at[dstv.at[j]], semd.at[k]).wait()
    plsc.subcore_barrier()
    pltpu.sync_copy(degacc.at[pl.ds(s * ROWS_PER_SUB, ROWS_PER_SUB)], zerov)
    pltpu.sync_copy(zerov, degp.at[c, pl.ds(s * ROWS_PER_SUB, ROWS_PER_SUB)])


_deg_call = pl.kernel(
    _deg_body,
    out_type=jax.ShapeDtypeStruct((NC, N_PAD), jnp.float32),
    mesh=_MESH,
    scratch_types=[
        pltpu.VMEM((CPTD, CHUNK), jnp.int32),
        pltpu.VMEM((CHUNK,), jnp.float32),
        pltpu.VMEM((ROWS_PER_SUB,), jnp.float32),
        pltpu.VMEM_SHARED((N_PAD,), jnp.float32),
        pltpu.SemaphoreType.DMA((NBD,)),
    ],
)


NBUF = 4
LAG = 2
IGRP = 16
NIGRP = CPT // IGRP


def _seg_body(g0, g1, src2d, dst2d, z2d_h, p3, srcb, dstb, rowsv, acc, table,
              semg, sems, semis, semid):
    c = lax.axis_index("c")
    s = lax.axis_index("s")
    ebase = s * CPT

    pltpu.sync_copy(z2d_h, rowsv.at[0])
    for r in range(ROWS_PER_SUB // CHUNK):
        sl = pl.ds(s * ROWS_PER_SUB + r * CHUNK, CHUNK)
        pltpu.sync_copy(rowsv.at[0], acc.at[sl])

    stg = pl.ds(s * ROWS_PER_SUB, ROWS_PER_SUB)

    @pl.when(c == 0)
    def _():
        pltpu.sync_copy(g0.at[stg], table.at[stg])

    @pl.when(c == 1)
    def _():
        pltpu.sync_copy(g1.at[stg], table.at[stg])

    plsc.subcore_barrier()

    pltpu.async_copy(src2d.at[pl.ds(ebase, IGRP)], srcb.at[0], semis.at[0])
    pltpu.async_copy(dst2d.at[pl.ds(ebase, IGRP)], dstb.at[0], semid.at[0])
    pltpu.make_async_copy(
        src2d.at[pl.ds(ebase, IGRP)], srcb.at[0], semis.at[0]).wait()
    pltpu.make_async_copy(
        dst2d.at[pl.ds(ebase, IGRP)], dstb.at[0], semid.at[0]).wait()
    for b in range(LAG):
        pltpu.async_copy(table.at[srcb.at[0, b]], rowsv.at[b], semg.at[b])

    def grp(gi, carry):
        bi = lax.rem(gi, 2)
        bn = lax.rem(gi + 1, 2)

        for k in range(IGRP):
            j = gi * IGRP + k
            b = k % NBUF
            bf = (b + LAG) % NBUF
            pltpu.make_async_copy(
                table.at[srcb.at[bi, k]], rowsv.at[b], semg.at[b]).wait()
            pltpu.async_copy(rowsv.at[b], acc.at[dstb.at[bi, k]], sems.at[b],
                             add=True)
            if k >= LAG:
                pltpu.make_async_copy(
                    rowsv.at[bf], acc.at[dstb.at[bi, k - LAG]],
                    sems.at[bf]).wait()
            else:
                @pl.when(j >= LAG)
                def _():
                    pltpu.make_async_copy(
                        rowsv.at[bf], acc.at[dstb.at[bn, k + IGRP - LAG]],
                        sems.at[bf]).wait()

            if k == LAG:
                @pl.when(gi + 1 < NIGRP)
                def _():
                    nb = ebase + (gi + 1) * IGRP
                    pltpu.async_copy(src2d.at[pl.ds(nb, IGRP)], srcb.at[bn],
                                     semis.at[bn])
                    pltpu.async_copy(dst2d.at[pl.ds(nb, IGRP)], dstb.at[bn],
                                     semid.at[bn])

            if k == IGRP - LAG:
                @pl.when(gi + 1 < NIGRP)
                def _():
                    nb = ebase + (gi + 1) * IGRP
                    pltpu.make_async_copy(
                        src2d.at[pl.ds(nb, IGRP)], srcb.at[bn],
                        semis.at[bn]).wait()
                    pltpu.make_async_copy(
                        dst2d.at[pl.ds(nb, IGRP)], dstb.at[bn],
                        semid.at[bn]).wait()

            if k + LAG < IGRP:
                pltpu.async_copy(table.at[srcb.at[bi, k + LAG]],
                                 rowsv.at[bf], semg.at[bf])
            else:
                @pl.when(gi + 1 < NIGRP)
                def _():
                    pltpu.async_copy(table.at[srcb.at[bn, k + LAG - IGRP]],
                                     rowsv.at[bf], semg.at[bf])
        return carry

    lax.fori_loop(0, NIGRP, grp, 0)
    for k in range(IGRP - LAG, IGRP):
        pltpu.make_async_copy(
            rowsv.at[k % NBUF], acc.at[dstb.at[(NIGRP - 1) % 2, k]],
            sems.at[k % NBUF]).wait()

    plsc.subcore_barrier()
    pltpu.sync_copy(acc.at[stg], p3.at[c, stg])


_seg_call = pl.kernel(
    _seg_body,
    out_type=jax.ShapeDtypeStruct((NC, N_PAD, DH), jnp.float32),
    mesh=_MESH,
    scratch_types=[
        pltpu.VMEM((2, IGRP, CHUNK), jnp.int32),
        pltpu.VMEM((2, IGRP, CHUNK), jnp.int32),
        pltpu.VMEM((NBUF, CHUNK, DH), jnp.float32),
        pltpu.VMEM_SHARED((N_PAD, DH), jnp.float32),
        pltpu.VMEM_SHARED((N_PAD, DH), jnp.float32),
        pltpu.SemaphoreType.DMA((NBUF,)),
        pltpu.SemaphoreType.DMA((NBUF,)),
        pltpu.SemaphoreType.DMA((2,)),
        pltpu.SemaphoreType.DMA((2,)),
    ],
    compiler_params=pltpu.CompilerParams(use_tc_tiling_on_sc=False),
)


_RB = 512

def _g_body(xref, wref, degref, g0ref, g1ref):
    deg = jnp.maximum(degref[0] + degref[1], 1.0)
    dinv = lax.rsqrt(deg)
    h = jnp.dot(xref[...], wref[...], preferred_element_type=jnp.float32) * dinv
    g0ref[...] = h[:, :DH]
    g1ref[...] = h[:, DH:]


def _g_call(x_pad, W, degp3):
    return pl.pallas_call(
        _g_body,
        grid=(N_PAD // _RB,),
        in_specs=[
            pl.BlockSpec((_RB, D), lambda i: (i, 0)),
            pl.BlockSpec((D, D), lambda i: (0, 0)),
            pl.BlockSpec((NC, _RB, 1), lambda i: (0, i, 0)),
        ],
        out_specs=[
            pl.BlockSpec((_RB, DH), lambda i: (i, 0)),
            pl.BlockSpec((_RB, DH), lambda i: (i, 0)),
        ],
        out_shape=[
            jax.ShapeDtypeStruct((N_PAD, DH), jnp.float32),
            jax.ShapeDtypeStruct((N_PAD, DH), jnp.float32),
        ],
    )(x_pad, W, degp3)


def _out_body(pref, degref, bref, oref):
    deg = jnp.maximum(degref[0] + degref[1], 1.0)
    dinv = lax.rsqrt(deg)
    s = jnp.concatenate([pref[0], pref[1]], axis=1)
    oref[...] = s * dinv + bref[...]


def _out_call(p3, degp3, b2d):
    return pl.pallas_call(
        _out_body,
        grid=(N_PAD // _RB,),
        in_specs=[
            pl.BlockSpec((NC, _RB, DH), lambda i: (0, i, 0)),
            pl.BlockSpec((NC, _RB, 1), lambda i: (0, i, 0)),
            pl.BlockSpec((1, D), lambda i: (0, 0)),
        ],
        out_specs=pl.BlockSpec((_RB, D), lambda i: (i, 0)),
        out_shape=jax.ShapeDtypeStruct((N_PAD, D), jnp.float32),
    )(p3, degp3, b2d)


@jax.jit
def kernel(t, x, edge_index, W, b):
    del t
    src = edge_index[0].astype(jnp.int32)
    dst = edge_index[1].astype(jnp.int32)
    pad = E_PAD - N_EDGES
    src2d = jnp.pad(src, (0, pad), constant_values=N_NODES).reshape(E_PAD // CHUNK, CHUNK)
    dst2d = jnp.pad(dst, (0, pad), constant_values=N_NODES).reshape(E_PAD // CHUNK, CHUNK)
    x_pad = jnp.pad(x.astype(jnp.float32), ((0, N_PAD - N_NODES), (0, 0)))

    ones_h = jnp.ones((CHUNK,), jnp.float32)
    zeros_h = jnp.zeros((ROWS_PER_SUB,), jnp.float32)
    z2d_h = jnp.zeros((CHUNK, DH), jnp.float32)

    degp = _deg_call(dst2d, ones_h, zeros_h)
    degp3 = degp.reshape(NC, N_PAD, 1)
    g0, g1 = _g_call(x_pad, W.astype(jnp.float32), degp3)
    p3 = _seg_call(g0, g1, src2d, dst2d, z2d_h)
    out = _out_call(p3, degp3, b.reshape(1, D).astype(jnp.float32))
    return out[:N_NODES]
